# baseline (speedup 1.0000x reference)
import functools

import jax
import jax.numpy as jnp
from jax import lax
from jax.experimental import pallas as pl
from jax.experimental.pallas import tpu as pltpu
from jax.experimental.pallas import tpu_sc as plsc

_N = 10000
_E = 320000
_D_IN = 128
_H = 64
_MAX_DEG = 10
_PRED_H = 128
_B = 64

_NC = 2
_NS = 16
_NW = _NC * _NS
_EW = _E // _NW
_CH = 80
_NCH = _EW // _CH
_NPAD = 10240
_RT = _NPAD // _NS

_BLK = 512
_NB = (_N + _BLK - 1) // _BLK


def _sc_zero16():
    return jnp.zeros((16,), jnp.float32)


def _sc_pipeline(table, srcv, dstv, rowsv, accsh, gsems, ssems, nch):
    def start_gather(i, b):
        pltpu.async_copy(table.at[srcv.at[i]], rowsv.at[b], gsems[b])

    def wait_gather(i, b):
        pltpu.make_async_copy(table.at[srcv.at[i]], rowsv.at[b],
                              gsems[b]).wait()

    def start_scatter(i, b):
        pltpu.async_copy(rowsv.at[b], accsh.at[dstv.at[i]], ssems[b],
                         add=True)

    def wait_scatter(i, b):
        pltpu.make_async_copy(rowsv.at[b], accsh.at[dstv.at[i]],
                              ssems[b]).wait()

    start_gather(0, 0)
    start_gather(1, 1)
    wait_gather(0, 0)
    start_scatter(0, 0)
    start_gather(2, 2)
    wait_gather(1, 1)
    start_scatter(1, 1)
    start_gather(3, 3)

    def group(g, _):
        for b in range(4):
            i = g * 4 + b + 2
            bb = (b + 2) % 4

            @pl.when(i < nch)
            def _():
                wait_gather(i, bb)
                start_scatter(i, bb)

            @pl.when(i - 2 < nch)
            def _():
                wait_scatter(i - 2, b)

                @pl.when(i + 2 < nch)
                def _():
                    start_gather(i + 2, b)
        return 0

    lax.fori_loop(0, (nch + 3) // 4, group, 0)


def _sc_zero_acc(rowsv0, accsh, s):
    z16 = jnp.zeros((16,), jnp.float32)
    d = rowsv0.shape[1]

    def zr(r, _):
        for k in range(d // 16):
            rowsv0[r, pl.ds(k * 16, 16)] = z16
        return 0

    lax.fori_loop(0, _CH, zr, 0)
    for q in range(_RT // _CH):
        pltpu.sync_copy(rowsv0, accsh.at[pl.ds(s * _RT + q * _CH, _CH)])


def _sc_zero_deg(degv):
    z16 = jnp.zeros((16,), jnp.float32)

    def zd(j, _):
        degv[pl.ds(j * 16, 16)] = z16
        return 0

    lax.fori_loop(0, _NPAD // 16, zd, 0)


def _sc_count_deg(dstv, degv, nch):
    ones16 = jnp.full((16,), 1.0, jnp.float32)

    def deg_row(j, _):
        for k in range(_CH // 16):
            d16 = dstv[j, pl.ds(k * 16, 16)]
            plsc.addupdate_scatter(degv, [d16], ones16)
        return 0

    lax.fori_loop(0, nch, deg_row, 0)


_SC_PARAMS = pltpu.CompilerParams(needs_layout_passes=False,
                                  use_tc_tiling_on_sc=False)
_NCH1 = _E // _NS // _CH


def _make_sc_agg1():
    mesh = plsc.VectorSubcoreMesh(core_axis_name="c", subcore_axis_name="s")
    D = _D_IN // 2
    out_type = [jax.ShapeDtypeStruct((_NC, _NPAD, D), jnp.float32),
                jax.ShapeDtypeStruct((_NS, _NPAD), jnp.float32)]
    scratch = [
        pltpu.VMEM((_NCH1, _CH), jnp.int32),
        pltpu.VMEM((_NCH1, _CH), jnp.int32),
        pltpu.VMEM((4, _CH, D), jnp.float32),
        pltpu.VMEM((_NPAD,), jnp.float32),
        pltpu.VMEM_SHARED((_NPAD, D), jnp.float32),
        [pltpu.SemaphoreType.DMA] * 4,
        [pltpu.SemaphoreType.DMA] * 4,
    ]

    def body(table, src4, dst3, agg_out, deg_out,
             srcv, dstv, rowsv, degv, accsh, gsems, ssems):
        c = lax.axis_index("c")
        s = lax.axis_index("s")

        pltpu.sync_copy(src4.at[c, s], srcv)
        pltpu.sync_copy(dst3.at[s], dstv)
        _sc_zero_acc(rowsv.at[0], accsh, s)

        @pl.when(c == 0)
        def _():
            _sc_zero_deg(degv)
            _sc_count_deg(dstv, degv, _NCH1)

        plsc.subcore_barrier()
        _sc_pipeline(table, srcv, dstv, rowsv, accsh, gsems, ssems, _NCH1)
        plsc.subcore_barrier()

        pltpu.sync_copy(accsh.at[pl.ds(s * _RT, _RT)],
                        agg_out.at[c, pl.ds(s * _RT, _RT)])

        @pl.when(c == 0)
        def _():
            pltpu.sync_copy(degv, deg_out.at[s])

    return pl.kernel(body, out_type=out_type, mesh=mesh,
                     scratch_types=scratch, compiler_params=_SC_PARAMS)


def _make_sc_agg2():
    mesh = plsc.VectorSubcoreMesh(core_axis_name="c", subcore_axis_name="s")
    out_type = [jax.ShapeDtypeStruct((_NC, _NPAD, _H), jnp.float32)]
    scratch = [
        pltpu.VMEM((_NCH, _CH), jnp.int32),
        pltpu.VMEM((_NCH, _CH), jnp.int32),
        pltpu.VMEM((4, _CH, _H), jnp.float32),
        pltpu.VMEM_SHARED((_NPAD, _H), jnp.float32),
        [pltpu.SemaphoreType.DMA] * 4,
        [pltpu.SemaphoreType.DMA] * 4,
    ]

    def body(table, src3, dst3, agg_out,
             srcv, dstv, rowsv, accsh, gsems, ssems):
        c = lax.axis_index("c")
        s = lax.axis_index("s")
        wid = s * _NC + c

        pltpu.sync_copy(src3.at[wid], srcv)
        pltpu.sync_copy(dst3.at[wid], dstv)
        _sc_zero_acc(rowsv.at[0], accsh, s)

        plsc.subcore_barrier()
        _sc_pipeline(table, srcv, dstv, rowsv, accsh, gsems, ssems, _NCH)
        plsc.subcore_barrier()

        pltpu.sync_copy(accsh.at[pl.ds(s * _RT, _RT)],
                        agg_out.at[c, pl.ds(s * _RT, _RT)])

    return pl.kernel(body, out_type=out_type, mesh=mesh,
                     scratch_types=scratch, compiler_params=_SC_PARAMS)


def _valid_mask(i):
    return (i * _BLK + lax.broadcasted_iota(jnp.int32, (_BLK, 1), 0)) < _N


def _deg_col(degp):
    ones = jnp.ones((_NS, 1), jnp.float32)
    return lax.dot_general(degp, ones, (((0,), (0,)), ((), ())),
                           preferred_element_type=jnp.float32)


def _bank_select(hwide, deg_col, width):
    degc = jnp.minimum(deg_col, float(_MAX_DEG))
    acc = jnp.zeros((_BLK, width), jnp.float32)
    for d in range(_MAX_DEG + 1):
        sel = degc == float(d)
        acc = acc + jnp.where(sel, hwide[:, d * width:(d + 1) * width], 0.0)
    return acc


def _affine(stats, gamma, beta):
    mean = stats[0:1, :] / float(_N)
    var = stats[1:2, :] / float(_N) - mean * mean
    a = gamma * lax.rsqrt(var + 1e-5)
    c = beta - mean * a
    return a, c


def _k1_body(feats, a0, a1, degp, w, b, h_out, st_out):
    i = pl.program_id(0)
    x = feats[...] + jnp.concatenate([a0[...], a1[...]], axis=1)
    deg = _deg_col(degp[...])
    hwide = jnp.dot(x.astype(jnp.bfloat16), w[...],
                    preferred_element_type=jnp.float32) + b[...]
    y = jnp.maximum(_bank_select(hwide, deg, _H), 0.0)
    valid = _valid_mask(i)
    ym = jnp.where(valid, y, 0.0)
    h_out[...] = y

    @pl.when(i == 0)
    def _():
        st_out[...] = jnp.zeros_like(st_out)

    st_out[...] += jnp.concatenate(
        [jnp.sum(ym, axis=0, keepdims=True),
         jnp.sum(jnp.where(valid, y * y, 0.0), axis=0, keepdims=True)], axis=0)


def _k2_body(h1, a0, a1, degp, st1, gamma1, beta1, w, b, h_out, st_out):
    i = pl.program_id(0)
    a_, c_ = _affine(st1[...], gamma1[...], beta1[...])
    deg = _deg_col(degp[...])
    x = a_ * (h1[...] + a0[...] + a1[...]) + c_ * (1.0 + deg)
    hwide = jnp.dot(x.astype(jnp.bfloat16), w[...],
                    preferred_element_type=jnp.float32) + b[...]
    y = jnp.maximum(_bank_select(hwide, deg, _H), 0.0)
    valid = _valid_mask(i)
    ym = jnp.where(valid, y, 0.0)
    h_out[...] = y

    @pl.when(i == 0)
    def _():
        st_out[...] = jnp.zeros_like(st_out)

    st_out[...] += jnp.concatenate(
        [jnp.sum(ym, axis=0, keepdims=True),
         jnp.sum(jnp.where(valid, y * y, 0.0), axis=0, keepdims=True)], axis=0)


def _k3_body(h2, ids, st2, gamma2, beta2, w, b,
             psum, pcnt, pmax, pmin, st_out):
    i = pl.program_id(0)
    a_, c_ = _affine(st2[...], gamma2[...], beta2[...])
    z = a_ * h2[...] + c_
    h3 = jnp.dot(z, w[...], preferred_element_type=jnp.float32) + b[...]
    valid = _valid_mask(i)
    h3 = jnp.where(valid, h3, 0.0)

    @pl.when(i == 0)
    def _():
        psum[...] = jnp.zeros_like(psum)
        pcnt[...] = jnp.zeros_like(pcnt)
        pmax[...] = jnp.full_like(pmax, -jnp.inf)
        pmin[...] = jnp.full_like(pmin, jnp.inf)
        st_out[...] = jnp.zeros_like(st_out)

    st_out[...] += jnp.concatenate(
        [jnp.sum(h3, axis=0, keepdims=True),
         jnp.sum(jnp.where(valid, h3 * h3, 0.0), axis=0, keepdims=True)],
        axis=0)

    idc = ids[...]
    onehot = jnp.where(
        (idc == lax.broadcasted_iota(jnp.int32, (1, _B), 1).astype(jnp.float32))
        & valid,
        1.0, 0.0)
    psum[...] += lax.dot_general(onehot, h3, (((0,), (0,)), ((), ())),
                                 preferred_element_type=jnp.float32)
    pcnt[...] += lax.dot_general(onehot, jnp.where(valid, 1.0, 0.0),
                                 (((0,), (0,)), ((), ())),
                                 preferred_element_type=jnp.float32)

    lo = jnp.min(jnp.where(valid, idc, float(_B))).astype(jnp.int32)
    hi = jnp.max(jnp.where(valid, idc, -1.0)).astype(jnp.int32)

    def mbody(g, _):
        m = (idc == g.astype(jnp.float32)) & valid
        row = lax.broadcasted_iota(jnp.int32, (_B, 1), 0) == g
        cmx = jnp.max(jnp.where(m, h3, -jnp.inf), axis=0, keepdims=True)
        cmn = jnp.min(jnp.where(m, h3, jnp.inf), axis=0, keepdims=True)
        pmax[...] = jnp.where(row, jnp.maximum(pmax[...], cmx), pmax[...])
        pmin[...] = jnp.where(row, jnp.minimum(pmin[...], cmn), pmin[...])
        return 0

    lax.fori_loop(lo, hi + 1, mbody, 0)


def _k4_body(psum, pcnt, pmax, pmin, st3, gamma_p, beta_p, w, b, out):
    a_, c_ = _affine(st3[...], gamma_p[...], beta_p[...])
    sum_n = a_ * psum[...] + c_ * pcnt[...]
    max_n = jnp.where(a_ >= 0.0, a_ * pmax[...] + c_, a_ * pmin[...] + c_)
    g = jnp.tanh(jnp.concatenate([sum_n, max_n], axis=1))
    out[...] = jnp.dot(g, w[...], preferred_element_type=jnp.float32) + b[...]


def _node_spec(width):
    return pl.BlockSpec((_BLK, width), lambda i: (i, 0))


def _const_spec(shape):
    return pl.BlockSpec(shape, lambda i: tuple(0 for _ in shape))


def kernel(feats, edge_index, node_graph_ids, W1, b1, gamma1, beta1,
           W2, b2, gamma2, beta2, W_ng, b_ng, gamma_p, beta_p, W_out, b_out):
    f32 = jnp.float32
    src = edge_index[0]
    dst = edge_index[1]
    src3 = src.reshape(_NW, _NCH, _CH)
    dst3 = dst.reshape(_NW, _NCH, _CH)
    src4 = jnp.stack([src, src + _N]).reshape(_NC, _NS, _NCH1, _CH)
    dst3a = dst.reshape(_NS, _NCH1, _CH)
    feats2 = jnp.concatenate([feats[:, :_D_IN // 2], feats[:, _D_IN // 2:]],
                             axis=0)
    ids_col = node_graph_ids.astype(f32)[:, None]
    w1t = W1.reshape((_MAX_DEG + 1) * _H, _D_IN).T.astype(jnp.bfloat16)
    b1r = b1.reshape(1, (_MAX_DEG + 1) * _H)
    w2t = W2.reshape((_MAX_DEG + 1) * _H, _H).T.astype(jnp.bfloat16)
    b2r = b2.reshape(1, (_MAX_DEG + 1) * _H)
    wngt = W_ng.T
    bngr = b_ng.reshape(1, _PRED_H)
    woutt = W_out.T
    boutr = b_out.reshape(1, 1)
    g1 = gamma1.reshape(1, _H)
    be1 = beta1.reshape(1, _H)
    g2 = gamma2.reshape(1, _H)
    be2 = beta2.reshape(1, _H)
    gp = gamma_p.reshape(1, _PRED_H)
    bep = beta_p.reshape(1, _PRED_H)

    agg1p, degp = _make_sc_agg1()(feats2, src4, dst3a)

    wide = (_MAX_DEG + 1) * _H
    h1p, st1 = pl.pallas_call(
        _k1_body,
        grid=(_NB,),
        in_specs=[
            _node_spec(_D_IN), _node_spec(_H), _node_spec(_H),
            pl.BlockSpec((_NS, _BLK), lambda i: (0, i)),
            _const_spec((_D_IN, wide)), _const_spec((1, wide)),
        ],
        out_specs=[_node_spec(_H), _const_spec((2, _H))],
        out_shape=[jax.ShapeDtypeStruct((_N, _H), f32),
                   jax.ShapeDtypeStruct((2, _H), f32)],
    )(feats, agg1p[0], agg1p[1], degp, w1t, b1r)

    agg2p = _make_sc_agg2()(h1p, src3, dst3)[0]

    h2p, st2 = pl.pallas_call(
        _k2_body,
        grid=(_NB,),
        in_specs=[
            _node_spec(_H), _node_spec(_H), _node_spec(_H),
            pl.BlockSpec((_NS, _BLK), lambda i: (0, i)),
            _const_spec((2, _H)), _const_spec((1, _H)), _const_spec((1, _H)),
            _const_spec((_H, wide)), _const_spec((1, wide)),
        ],
        out_specs=[_node_spec(_H), _const_spec((2, _H))],
        out_shape=[jax.ShapeDtypeStruct((_N, _H), f32),
                   jax.ShapeDtypeStruct((2, _H), f32)],
    )(h1p, agg2p[0], agg2p[1], degp, st1, g1, be1, w2t, b2r)

    psum, pcnt, pmax, pmin, st3 = pl.pallas_call(
        _k3_body,
        grid=(_NB,),
        in_specs=[
            _node_spec(_H), _node_spec(1),
            _const_spec((2, _H)), _const_spec((1, _H)), _const_spec((1, _H)),
            _const_spec((_H, _PRED_H)), _const_spec((1, _PRED_H)),
        ],
        out_specs=[_const_spec((_B, _PRED_H)), _const_spec((_B, 1)),
                   _const_spec((_B, _PRED_H)), _const_spec((_B, _PRED_H)),
                   _const_spec((2, _PRED_H))],
        out_shape=[jax.ShapeDtypeStruct((_B, _PRED_H), f32),
                   jax.ShapeDtypeStruct((_B, 1), f32),
                   jax.ShapeDtypeStruct((_B, _PRED_H), f32),
                   jax.ShapeDtypeStruct((_B, _PRED_H), f32),
                   jax.ShapeDtypeStruct((2, _PRED_H), f32)],
    )(h2p, ids_col, st2, g2, be2, wngt, bngr)

    out = pl.pallas_call(
        _k4_body,
        out_shape=jax.ShapeDtypeStruct((_B, 1), f32),
    )(psum, pcnt, pmax, pmin, st3, gp, bep, woutt, boutr)
    return out


# speedup vs baseline: 1.0987x; 1.0987x over previous
import functools

import jax
import jax.numpy as jnp
from jax import lax
from jax.experimental import pallas as pl
from jax.experimental.pallas import tpu as pltpu
from jax.experimental.pallas import tpu_sc as plsc

_N = 10000
_E = 320000
_D_IN = 128
_H = 64
_MAX_DEG = 10
_PRED_H = 128
_B = 64

_NC = 2
_NS = 16
_NW = _NC * _NS
_EW = _E // _NW
_CH = 80
_NCH = _EW // _CH
_NPAD = 10240
_RT = _NPAD // _NS
_RT1 = _N // _NS

_BLK = 512
_NB = (_N + _BLK - 1) // _BLK


def _sc_zero16():
    return jnp.zeros((16,), jnp.float32)


def _sc_pipeline(table, srcv, dstv, rowsv, accsh, gsems, ssems, nch):
    def start_gather(i, b):
        pltpu.async_copy(table.at[srcv.at[i]], rowsv.at[b], gsems[b])

    def wait_gather(i, b):
        pltpu.make_async_copy(table.at[srcv.at[i]], rowsv.at[b],
                              gsems[b]).wait()

    def start_scatter(i, b):
        pltpu.async_copy(rowsv.at[b], accsh.at[dstv.at[i]], ssems[b],
                         add=True)

    def wait_scatter(i, b):
        pltpu.make_async_copy(rowsv.at[b], accsh.at[dstv.at[i]],
                              ssems[b]).wait()

    start_gather(0, 0)
    start_gather(1, 1)
    wait_gather(0, 0)
    start_scatter(0, 0)
    start_gather(2, 2)
    wait_gather(1, 1)
    start_scatter(1, 1)
    start_gather(3, 3)

    def group(g, _):
        for b in range(4):
            i = g * 4 + b + 2
            bb = (b + 2) % 4

            @pl.when(i < nch)
            def _():
                wait_gather(i, bb)
                start_scatter(i, bb)

            @pl.when(i - 2 < nch)
            def _():
                wait_scatter(i - 2, b)

                @pl.when(i + 2 < nch)
                def _():
                    start_gather(i + 2, b)
        return 0

    lax.fori_loop(0, (nch + 3) // 4, group, 0)


def _sc_zero_acc(rowsv0, accsh, s):
    z16 = jnp.zeros((16,), jnp.float32)
    d = rowsv0.shape[1]

    def zr(r, _):
        for k in range(d // 16):
            rowsv0[r, pl.ds(k * 16, 16)] = z16
        return 0

    lax.fori_loop(0, _CH, zr, 0)
    for q in range(_RT1 // _CH):
        pltpu.sync_copy(rowsv0, accsh.at[pl.ds(s * _RT1 + q * _CH, _CH)])
    rem = _RT1 % _CH
    if rem:
        pltpu.sync_copy(
            rowsv0.at[pl.ds(0, rem)],
            accsh.at[pl.ds(s * _RT1 + _RT1 - rem, rem)])


def _sc_zero_deg(degv):
    z16 = jnp.zeros((16,), jnp.float32)

    def zd(j, _):
        degv[pl.ds(j * 16, 16)] = z16
        return 0

    lax.fori_loop(0, _N // 16, zd, 0)


def _sc_count_deg(dstv, degv, nch):
    ones16 = jnp.full((16,), 1.0, jnp.float32)

    def deg_row(j, _):
        for k in range(_CH // 16):
            d16 = dstv[j, pl.ds(k * 16, 16)]
            plsc.addupdate_scatter(degv, [d16], ones16)
        return 0

    lax.fori_loop(0, nch, deg_row, 0)


_SC_PARAMS = pltpu.CompilerParams(needs_layout_passes=False,
                                  use_tc_tiling_on_sc=False)
_NCH1 = _E // _NS // _CH


def _make_sc_agg1():
    mesh = plsc.VectorSubcoreMesh(core_axis_name="c", subcore_axis_name="s")
    D = _D_IN // 2
    out_type = [jax.ShapeDtypeStruct((_NC, _N, D), jnp.float32),
                jax.ShapeDtypeStruct((_NS, _N), jnp.float32)]
    scratch = [
        pltpu.VMEM((_NCH1, _CH), jnp.int32),
        pltpu.VMEM((_NCH1, _CH), jnp.int32),
        pltpu.VMEM((_NCH1, _CH), jnp.int32),
        pltpu.VMEM((4, _CH, D), jnp.float32),
        pltpu.VMEM((_N,), jnp.float32),
        pltpu.VMEM_SHARED((_N, D), jnp.float32),
        [pltpu.SemaphoreType.DMA] * 4,
        [pltpu.SemaphoreType.DMA] * 4,
    ]

    def body(table, src3, dst3, agg_out, deg_out,
             srcv, srci, dstv, rowsv, degv, accsh, gsems, ssems):
        c = lax.axis_index("c")
        s = lax.axis_index("s")

        pltpu.sync_copy(src3.at[s], srcv)
        pltpu.sync_copy(dst3.at[s], dstv)

        def dec_row(r, _):
            for k in range(_CH // 16):
                v = srcv[r, pl.ds(k * 16, 16)]
                srci[r, pl.ds(k * 16, 16)] = v * 2 + c
            return 0

        lax.fori_loop(0, _NCH1, dec_row, 0)
        _sc_zero_acc(rowsv.at[0], accsh, s)

        @pl.when(c == 0)
        def _():
            _sc_zero_deg(degv)
            _sc_count_deg(dstv, degv, _NCH1)

        plsc.subcore_barrier()
        _sc_pipeline(table, srci, dstv, rowsv, accsh, gsems, ssems, _NCH1)
        plsc.subcore_barrier()

        pltpu.sync_copy(accsh.at[pl.ds(s * _RT1, _RT1)],
                        agg_out.at[c, pl.ds(s * _RT1, _RT1)])

        @pl.when(c == 0)
        def _():
            pltpu.sync_copy(degv, deg_out.at[s])

    return pl.kernel(body, out_type=out_type, mesh=mesh,
                     scratch_types=scratch, compiler_params=_SC_PARAMS)


def _make_sc_agg2():
    mesh = plsc.VectorSubcoreMesh(core_axis_name="c", subcore_axis_name="s")
    out_type = [jax.ShapeDtypeStruct((_NC, _N, _H), jnp.float32)]
    scratch = [
        pltpu.VMEM((_NCH, _CH), jnp.int32),
        pltpu.VMEM((_NCH, _CH), jnp.int32),
        pltpu.VMEM((4, _CH, _H), jnp.float32),
        pltpu.VMEM_SHARED((_N, _H), jnp.float32),
        [pltpu.SemaphoreType.DMA] * 4,
        [pltpu.SemaphoreType.DMA] * 4,
    ]

    def body(table, src3, dst3, agg_out,
             srcv, dstv, rowsv, accsh, gsems, ssems):
        c = lax.axis_index("c")
        s = lax.axis_index("s")
        wid = s * _NC + c

        pltpu.sync_copy(src3.at[wid], srcv)
        pltpu.sync_copy(dst3.at[wid], dstv)
        _sc_zero_acc(rowsv.at[0], accsh, s)

        plsc.subcore_barrier()
        _sc_pipeline(table, srcv, dstv, rowsv, accsh, gsems, ssems, _NCH)
        plsc.subcore_barrier()

        pltpu.sync_copy(accsh.at[pl.ds(s * _RT1, _RT1)],
                        agg_out.at[c, pl.ds(s * _RT1, _RT1)])

    return pl.kernel(body, out_type=out_type, mesh=mesh,
                     scratch_types=scratch, compiler_params=_SC_PARAMS)


def _valid_mask(i):
    return (i * _BLK + lax.broadcasted_iota(jnp.int32, (_BLK, 1), 0)) < _N


def _deg_col(degp):
    ones = jnp.ones((_NS, 1), jnp.float32)
    return lax.dot_general(degp, ones, (((0,), (0,)), ((), ())),
                           preferred_element_type=jnp.float32)


def _bank_select(hwide, deg_col, width):
    degc = jnp.minimum(deg_col, float(_MAX_DEG))
    acc = jnp.zeros((_BLK, width), jnp.float32)
    for d in range(_MAX_DEG + 1):
        sel = degc == float(d)
        acc = acc + jnp.where(sel, hwide[:, d * width:(d + 1) * width], 0.0)
    return acc


def _affine(stats, gamma, beta):
    mean = stats[0:1, :] / float(_N)
    var = stats[1:2, :] / float(_N) - mean * mean
    a = gamma * lax.rsqrt(var + 1e-5)
    c = beta - mean * a
    return a, c


def _k1_body(feats, a0, a1, degp, w, b, h_out, st_out):
    i = pl.program_id(0)
    x = feats[...] + jnp.concatenate([a0[...], a1[...]], axis=1)
    deg = _deg_col(degp[...])
    hwide = jnp.dot(x.astype(jnp.bfloat16), w[...],
                    preferred_element_type=jnp.float32) + b[...]
    y = jnp.maximum(_bank_select(hwide, deg, _H), 0.0)
    valid = _valid_mask(i)
    ym = jnp.where(valid, y, 0.0)
    h_out[...] = y

    @pl.when(i == 0)
    def _():
        st_out[...] = jnp.zeros_like(st_out)

    st_out[...] += jnp.concatenate(
        [jnp.sum(ym, axis=0, keepdims=True),
         jnp.sum(jnp.where(valid, y * y, 0.0), axis=0, keepdims=True)], axis=0)


def _k2_body(h1, a0, a1, degp, st1, gamma1, beta1, w, b, h_out, st_out):
    i = pl.program_id(0)
    a_, c_ = _affine(st1[...], gamma1[...], beta1[...])
    deg = _deg_col(degp[...])
    x = a_ * (h1[...] + a0[...] + a1[...]) + c_ * (1.0 + deg)
    hwide = jnp.dot(x.astype(jnp.bfloat16), w[...],
                    preferred_element_type=jnp.float32) + b[...]
    y = jnp.maximum(_bank_select(hwide, deg, _H), 0.0)
    valid = _valid_mask(i)
    ym = jnp.where(valid, y, 0.0)
    h_out[...] = y

    @pl.when(i == 0)
    def _():
        st_out[...] = jnp.zeros_like(st_out)

    st_out[...] += jnp.concatenate(
        [jnp.sum(ym, axis=0, keepdims=True),
         jnp.sum(jnp.where(valid, y * y, 0.0), axis=0, keepdims=True)], axis=0)


def _k3_body(h2, ids, st2, gamma2, beta2, w, b,
             psum, pcnt, pmax, pmin, st_out):
    i = pl.program_id(0)
    a_, c_ = _affine(st2[...], gamma2[...], beta2[...])
    z = a_ * h2[...] + c_
    h3 = jnp.dot(z, w[...], preferred_element_type=jnp.float32) + b[...]
    valid = _valid_mask(i)
    h3 = jnp.where(valid, h3, 0.0)

    @pl.when(i == 0)
    def _():
        psum[...] = jnp.zeros_like(psum)
        pcnt[...] = jnp.zeros_like(pcnt)
        pmax[...] = jnp.full_like(pmax, -jnp.inf)
        pmin[...] = jnp.full_like(pmin, jnp.inf)
        st_out[...] = jnp.zeros_like(st_out)

    st_out[...] += jnp.concatenate(
        [jnp.sum(h3, axis=0, keepdims=True),
         jnp.sum(jnp.where(valid, h3 * h3, 0.0), axis=0, keepdims=True)],
        axis=0)

    idc = ids[...]
    onehot = jnp.where(
        (idc == lax.broadcasted_iota(jnp.int32, (1, _B), 1).astype(jnp.float32))
        & valid,
        1.0, 0.0)
    psum[...] += lax.dot_general(onehot, h3, (((0,), (0,)), ((), ())),
                                 preferred_element_type=jnp.float32)
    pcnt[...] += lax.dot_general(onehot, jnp.where(valid, 1.0, 0.0),
                                 (((0,), (0,)), ((), ())),
                                 preferred_element_type=jnp.float32)

    lo = jnp.min(jnp.where(valid, idc, float(_B))).astype(jnp.int32)
    hi = jnp.max(jnp.where(valid, idc, -1.0)).astype(jnp.int32)

    def mbody(g, _):
        m = (idc == g.astype(jnp.float32)) & valid
        row = lax.broadcasted_iota(jnp.int32, (_B, 1), 0) == g
        cmx = jnp.max(jnp.where(m, h3, -jnp.inf), axis=0, keepdims=True)
        cmn = jnp.min(jnp.where(m, h3, jnp.inf), axis=0, keepdims=True)
        pmax[...] = jnp.where(row, jnp.maximum(pmax[...], cmx), pmax[...])
        pmin[...] = jnp.where(row, jnp.minimum(pmin[...], cmn), pmin[...])
        return 0

    lax.fori_loop(lo, hi + 1, mbody, 0)


def _k4_body(psum, pcnt, pmax, pmin, st3, gamma_p, beta_p, w, b, out):
    a_, c_ = _affine(st3[...], gamma_p[...], beta_p[...])
    sum_n = a_ * psum[...] + c_ * pcnt[...]
    max_n = jnp.where(a_ >= 0.0, a_ * pmax[...] + c_, a_ * pmin[...] + c_)
    g = jnp.tanh(jnp.concatenate([sum_n, max_n], axis=1))
    out[...] = jnp.dot(g, w[...], preferred_element_type=jnp.float32) + b[...]


def _node_spec(width):
    return pl.BlockSpec((_BLK, width), lambda i: (i, 0))


def _const_spec(shape):
    return pl.BlockSpec(shape, lambda i: tuple(0 for _ in shape))


def kernel(feats, edge_index, node_graph_ids, W1, b1, gamma1, beta1,
           W2, b2, gamma2, beta2, W_ng, b_ng, gamma_p, beta_p, W_out, b_out):
    f32 = jnp.float32
    src = edge_index[0]
    dst = edge_index[1]
    src3 = src.reshape(_NW, _NCH, _CH)
    dst3 = dst.reshape(_NW, _NCH, _CH)
    src3a = src.reshape(_NS, _NCH1, _CH)
    dst3a = dst.reshape(_NS, _NCH1, _CH)
    table1 = feats.reshape(2 * _N, _D_IN // 2)
    ids_col = node_graph_ids.astype(f32)[:, None]
    w1t = W1.reshape((_MAX_DEG + 1) * _H, _D_IN).T.astype(jnp.bfloat16)
    b1r = b1.reshape(1, (_MAX_DEG + 1) * _H)
    w2t = W2.reshape((_MAX_DEG + 1) * _H, _H).T.astype(jnp.bfloat16)
    b2r = b2.reshape(1, (_MAX_DEG + 1) * _H)
    wngt = W_ng.T
    bngr = b_ng.reshape(1, _PRED_H)
    woutt = W_out.T
    boutr = b_out.reshape(1, 1)
    g1 = gamma1.reshape(1, _H)
    be1 = beta1.reshape(1, _H)
    g2 = gamma2.reshape(1, _H)
    be2 = beta2.reshape(1, _H)
    gp = gamma_p.reshape(1, _PRED_H)
    bep = beta_p.reshape(1, _PRED_H)

    agg1p, degp = _make_sc_agg1()(table1, src3a, dst3a)

    wide = (_MAX_DEG + 1) * _H
    h1p, st1 = pl.pallas_call(
        _k1_body,
        grid=(_NB,),
        in_specs=[
            _node_spec(_D_IN), _node_spec(_H), _node_spec(_H),
            pl.BlockSpec((_NS, _BLK), lambda i: (0, i)),
            _const_spec((_D_IN, wide)), _const_spec((1, wide)),
        ],
        out_specs=[_node_spec(_H), _const_spec((2, _H))],
        out_shape=[jax.ShapeDtypeStruct((_N, _H), f32),
                   jax.ShapeDtypeStruct((2, _H), f32)],
    )(feats, agg1p[0], agg1p[1], degp, w1t, b1r)

    agg2p = _make_sc_agg2()(h1p, src3, dst3)[0]

    h2p, st2 = pl.pallas_call(
        _k2_body,
        grid=(_NB,),
        in_specs=[
            _node_spec(_H), _node_spec(_H), _node_spec(_H),
            pl.BlockSpec((_NS, _BLK), lambda i: (0, i)),
            _const_spec((2, _H)), _const_spec((1, _H)), _const_spec((1, _H)),
            _const_spec((_H, wide)), _const_spec((1, wide)),
        ],
        out_specs=[_node_spec(_H), _const_spec((2, _H))],
        out_shape=[jax.ShapeDtypeStruct((_N, _H), f32),
                   jax.ShapeDtypeStruct((2, _H), f32)],
    )(h1p, agg2p[0], agg2p[1], degp, st1, g1, be1, w2t, b2r)

    psum, pcnt, pmax, pmin, st3 = pl.pallas_call(
        _k3_body,
        grid=(_NB,),
        in_specs=[
            _node_spec(_H), _node_spec(1),
            _const_spec((2, _H)), _const_spec((1, _H)), _const_spec((1, _H)),
            _const_spec((_H, _PRED_H)), _const_spec((1, _PRED_H)),
        ],
        out_specs=[_const_spec((_B, _PRED_H)), _const_spec((_B, 1)),
                   _const_spec((_B, _PRED_H)), _const_spec((_B, _PRED_H)),
                   _const_spec((2, _PRED_H))],
        out_shape=[jax.ShapeDtypeStruct((_B, _PRED_H), f32),
                   jax.ShapeDtypeStruct((_B, 1), f32),
                   jax.ShapeDtypeStruct((_B, _PRED_H), f32),
                   jax.ShapeDtypeStruct((_B, _PRED_H), f32),
                   jax.ShapeDtypeStruct((2, _PRED_H), f32)],
    )(h2p, ids_col, st2, g2, be2, wngt, bngr)

    out = pl.pallas_call(
        _k4_body,
        out_shape=jax.ShapeDtypeStruct((_B, 1), f32),
    )(psum, pcnt, pmax, pmin, st3, gp, bep, woutt, boutr)
    return out


# speedup vs baseline: 1.1005x; 1.0016x over previous
import functools

import jax
import jax.numpy as jnp
from jax import lax
from jax.experimental import pallas as pl
from jax.experimental.pallas import tpu as pltpu
from jax.experimental.pallas import tpu_sc as plsc

_N = 10000
_E = 320000
_D_IN = 128
_H = 64
_MAX_DEG = 10
_PRED_H = 128
_B = 64

_NC = 2
_NS = 16
_NW = _NC * _NS
_EW = _E // _NW
_CH = 80
_NCH = _EW // _CH
_RT1 = _N // _NS

_BLK = 512
_NB = (_N + _BLK - 1) // _BLK


def _sc_pipeline(table, srcv, dstv, rowsv, accsh, gsems, ssems, nch):
    def start_gather(i, b):
        pltpu.async_copy(table.at[srcv.at[i]], rowsv.at[b], gsems[b])

    def wait_gather(i, b):
        pltpu.make_async_copy(table.at[srcv.at[i]], rowsv.at[b],
                              gsems[b]).wait()

    def start_scatter(i, b):
        pltpu.async_copy(rowsv.at[b], accsh.at[dstv.at[i]], ssems[b],
                         add=True)

    def wait_scatter(i, b):
        pltpu.make_async_copy(rowsv.at[b], accsh.at[dstv.at[i]],
                              ssems[b]).wait()

    start_gather(0, 0)
    start_gather(1, 1)
    wait_gather(0, 0)
    start_scatter(0, 0)
    start_gather(2, 2)
    wait_gather(1, 1)
    start_scatter(1, 1)
    start_gather(3, 3)

    def group(g, _):
        for b in range(4):
            i = g * 4 + b + 2
            bb = (b + 2) % 4

            @pl.when(i < nch)
            def _():
                wait_gather(i, bb)
                start_scatter(i, bb)

            @pl.when(i - 2 < nch)
            def _():
                wait_scatter(i - 2, b)

                @pl.when(i + 2 < nch)
                def _():
                    start_gather(i + 2, b)
        return 0

    lax.fori_loop(0, (nch + 3) // 4, group, 0)


def _sc_zero_acc(rowsv0, accsh, s):
    z16 = jnp.zeros((16,), jnp.float32)
    d = rowsv0.shape[1]

    def zr(r, _):
        for k in range(d // 16):
            rowsv0[r, pl.ds(k * 16, 16)] = z16
        return 0

    lax.fori_loop(0, _CH, zr, 0)
    for q in range(_RT1 // _CH):
        pltpu.sync_copy(rowsv0, accsh.at[pl.ds(s * _RT1 + q * _CH, _CH)])
    rem = _RT1 % _CH
    if rem:
        pltpu.sync_copy(
            rowsv0.at[pl.ds(0, rem)],
            accsh.at[pl.ds(s * _RT1 + _RT1 - rem, rem)])


def _sc_zero_deg(degv):
    z16 = jnp.zeros((16,), jnp.float32)

    def zd(j, _):
        degv[pl.ds(j * 16, 16)] = z16
        return 0

    lax.fori_loop(0, _N // 16, zd, 0)


def _sc_count_deg(dstv, degv, nch):
    ones16 = jnp.full((16,), 1.0, jnp.float32)

    def deg_row(j, _):
        for k in range(_CH // 16):
            d16 = dstv[j, pl.ds(k * 16, 16)]
            plsc.addupdate_scatter(degv, [d16], ones16)
        return 0

    lax.fori_loop(0, nch, deg_row, 0)


_SC_PARAMS = pltpu.CompilerParams(needs_layout_passes=False,
                                  use_tc_tiling_on_sc=False)
_NCH1 = _E // _NS // _CH


def _make_sc_agg1():
    mesh = plsc.VectorSubcoreMesh(core_axis_name="c", subcore_axis_name="s")
    D = _D_IN // 2
    out_type = [jax.ShapeDtypeStruct((_NC, _N, D), jnp.float32),
                jax.ShapeDtypeStruct((_NS, _N), jnp.float32)]
    scratch = [
        pltpu.VMEM((_NCH1, _CH), jnp.int32),
        pltpu.VMEM((_NCH1, _CH), jnp.int32),
        pltpu.VMEM((_NCH1, _CH), jnp.int32),
        pltpu.VMEM((4, _CH, D), jnp.float32),
        pltpu.VMEM((_N,), jnp.float32),
        pltpu.VMEM_SHARED((_N, D), jnp.float32),
        [pltpu.SemaphoreType.DMA] * 4,
        [pltpu.SemaphoreType.DMA] * 4,
    ]

    def body(table, src3, dst3, agg_out, deg_out,
             srcv, srci, dstv, rowsv, degv, accsh, gsems, ssems):
        c = lax.axis_index("c")
        s = lax.axis_index("s")

        pltpu.sync_copy(src3.at[s], srcv)
        pltpu.sync_copy(dst3.at[s], dstv)

        def dec_row(r, _):
            for k in range(_CH // 16):
                v = srcv[r, pl.ds(k * 16, 16)]
                srci[r, pl.ds(k * 16, 16)] = v * 2 + c
            return 0

        lax.fori_loop(0, _NCH1, dec_row, 0)
        _sc_zero_acc(rowsv.at[0], accsh, s)

        @pl.when(c == 0)
        def _():
            _sc_zero_deg(degv)
            _sc_count_deg(dstv, degv, _NCH1)

        plsc.subcore_barrier()
        _sc_pipeline(table, srci, dstv, rowsv, accsh, gsems, ssems, _NCH1)
        plsc.subcore_barrier()

        pltpu.sync_copy(accsh.at[pl.ds(s * _RT1, _RT1)],
                        agg_out.at[c, pl.ds(s * _RT1, _RT1)])

        @pl.when(c == 0)
        def _():
            pltpu.sync_copy(degv, deg_out.at[s])

    return pl.kernel(body, out_type=out_type, mesh=mesh,
                     scratch_types=scratch, compiler_params=_SC_PARAMS)


def _make_sc_agg2():
    mesh = plsc.VectorSubcoreMesh(core_axis_name="c", subcore_axis_name="s")
    out_type = [jax.ShapeDtypeStruct((_NC, _N, _H), jnp.float32)]
    scratch = [
        pltpu.VMEM((_NCH, _CH), jnp.int32),
        pltpu.VMEM((_NCH, _CH), jnp.int32),
        pltpu.VMEM((4, _CH, _H), jnp.float32),
        pltpu.VMEM_SHARED((_N, _H), jnp.float32),
        [pltpu.SemaphoreType.DMA] * 4,
        [pltpu.SemaphoreType.DMA] * 4,
    ]

    def body(table, src3, dst3, agg_out,
             srcv, dstv, rowsv, accsh, gsems, ssems):
        c = lax.axis_index("c")
        s = lax.axis_index("s")
        wid = s * _NC + c

        pltpu.sync_copy(src3.at[wid], srcv)
        pltpu.sync_copy(dst3.at[wid], dstv)
        _sc_zero_acc(rowsv.at[0], accsh, s)

        plsc.subcore_barrier()
        _sc_pipeline(table, srcv, dstv, rowsv, accsh, gsems, ssems, _NCH)
        plsc.subcore_barrier()

        pltpu.sync_copy(accsh.at[pl.ds(s * _RT1, _RT1)],
                        agg_out.at[c, pl.ds(s * _RT1, _RT1)])

    return pl.kernel(body, out_type=out_type, mesh=mesh,
                     scratch_types=scratch, compiler_params=_SC_PARAMS)


def _valid_mask(i):
    return (i * _BLK + lax.broadcasted_iota(jnp.int32, (_BLK, 1), 0)) < _N


def _deg_col(degp):
    ones = jnp.ones((_NS, 1), jnp.float32)
    return lax.dot_general(degp, ones, (((0,), (0,)), ((), ())),
                           preferred_element_type=jnp.float32)


def _bank_select(hwide, deg_col, width):
    degc = jnp.minimum(deg_col, float(_MAX_DEG))
    acc = jnp.zeros((_BLK, width), jnp.float32)
    for d in range(_MAX_DEG + 1):
        sel = degc == float(d)
        acc = acc + jnp.where(sel, hwide[:, d * width:(d + 1) * width], 0.0)
    return acc


def _affine(stats, gamma, beta):
    mean = stats[0:1, :] / float(_N)
    var = stats[1:2, :] / float(_N) - mean * mean
    a = gamma * lax.rsqrt(var + 1e-5)
    c = beta - mean * a
    return a, c


def _k1_body(feats, a0, a1, degp, w, b, h_out, st_out):
    i = pl.program_id(0)
    x = feats[...] + jnp.concatenate([a0[...], a1[...]], axis=1)
    deg = _deg_col(degp[...])
    hwide = jnp.dot(x.astype(jnp.bfloat16), w[...],
                    preferred_element_type=jnp.float32) + b[...]
    y = jnp.maximum(_bank_select(hwide, deg, _H), 0.0)
    valid = _valid_mask(i)
    ym = jnp.where(valid, y, 0.0)
    h_out[...] = y

    @pl.when(i == 0)
    def _():
        st_out[...] = jnp.zeros_like(st_out)

    st_out[...] += jnp.concatenate(
        [jnp.sum(ym, axis=0, keepdims=True),
         jnp.sum(jnp.where(valid, y * y, 0.0), axis=0, keepdims=True)], axis=0)


def _k2_body(h1, a0, a1, degp, st1, gamma1, beta1, w, b, h_out, st_out):
    i = pl.program_id(0)
    a_, c_ = _affine(st1[...], gamma1[...], beta1[...])
    deg = _deg_col(degp[...])
    x = a_ * (h1[...] + a0[...] + a1[...]) + c_ * (1.0 + deg)
    hwide = jnp.dot(x.astype(jnp.bfloat16), w[...],
                    preferred_element_type=jnp.float32) + b[...]
    y = jnp.maximum(_bank_select(hwide, deg, _H), 0.0)
    valid = _valid_mask(i)
    ym = jnp.where(valid, y, 0.0)
    h_out[...] = y

    @pl.when(i == 0)
    def _():
        st_out[...] = jnp.zeros_like(st_out)

    st_out[...] += jnp.concatenate(
        [jnp.sum(ym, axis=0, keepdims=True),
         jnp.sum(jnp.where(valid, y * y, 0.0), axis=0, keepdims=True)], axis=0)


def _k3_body(h2, ids, st2, gamma2, beta2, w, b,
             psum, pcnt, pmax, pmin, st_out):
    i = pl.program_id(0)
    a_, c_ = _affine(st2[...], gamma2[...], beta2[...])
    z = a_ * h2[...] + c_
    h3 = jnp.dot(z, w[...], preferred_element_type=jnp.float32) + b[...]
    valid = _valid_mask(i)
    h3 = jnp.where(valid, h3, 0.0)

    @pl.when(i == 0)
    def _():
        psum[...] = jnp.zeros_like(psum)
        pcnt[...] = jnp.zeros_like(pcnt)
        pmax[...] = jnp.full_like(pmax, -jnp.inf)
        pmin[...] = jnp.full_like(pmin, jnp.inf)
        st_out[...] = jnp.zeros_like(st_out)

    st_out[...] += jnp.concatenate(
        [jnp.sum(h3, axis=0, keepdims=True),
         jnp.sum(jnp.where(valid, h3 * h3, 0.0), axis=0, keepdims=True)],
        axis=0)

    idc = ids[...]
    onehot = jnp.where(
        (idc == lax.broadcasted_iota(jnp.int32, (1, _B), 1).astype(jnp.float32))
        & valid,
        1.0, 0.0)
    psum[...] += lax.dot_general(onehot, h3, (((0,), (0,)), ((), ())),
                                 preferred_element_type=jnp.float32)
    pcnt[...] += lax.dot_general(onehot, jnp.where(valid, 1.0, 0.0),
                                 (((0,), (0,)), ((), ())),
                                 preferred_element_type=jnp.float32)

    lo = jnp.min(jnp.where(valid, idc, float(_B))).astype(jnp.int32)
    hi = jnp.max(jnp.where(valid, idc, -1.0)).astype(jnp.int32)

    def mbody(g, _):
        m = (idc == g.astype(jnp.float32)) & valid
        row = lax.broadcasted_iota(jnp.int32, (_B, 1), 0) == g
        cmx = jnp.max(jnp.where(m, h3, -jnp.inf), axis=0, keepdims=True)
        cmn = jnp.min(jnp.where(m, h3, jnp.inf), axis=0, keepdims=True)
        pmax[...] = jnp.where(row, jnp.maximum(pmax[...], cmx), pmax[...])
        pmin[...] = jnp.where(row, jnp.minimum(pmin[...], cmn), pmin[...])
        return 0

    lax.fori_loop(lo, hi + 1, mbody, 0)


def _k4_body(psum, pcnt, pmax, pmin, st3, gamma_p, beta_p, w, b, out):
    a_, c_ = _affine(st3[...], gamma_p[...], beta_p[...])
    sum_n = a_ * psum[...] + c_ * pcnt[...]
    max_n = jnp.where(a_ >= 0.0, a_ * pmax[...] + c_, a_ * pmin[...] + c_)
    g = jnp.tanh(jnp.concatenate([sum_n, max_n], axis=1))
    out[...] = jnp.dot(g, w[...], preferred_element_type=jnp.float32) + b[...]


def _node_spec(width):
    return pl.BlockSpec((_BLK, width), lambda i: (i, 0))


def _const_spec(shape):
    return pl.BlockSpec(shape, lambda i: tuple(0 for _ in shape))


def kernel(feats, edge_index, node_graph_ids, W1, b1, gamma1, beta1,
           W2, b2, gamma2, beta2, W_ng, b_ng, gamma_p, beta_p, W_out, b_out):
    f32 = jnp.float32
    src = edge_index[0]
    dst = edge_index[1]
    src3 = src.reshape(_NW, _NCH, _CH)
    dst3 = dst.reshape(_NW, _NCH, _CH)
    src3a = src.reshape(_NS, _NCH1, _CH)
    dst3a = dst.reshape(_NS, _NCH1, _CH)
    table1 = feats.reshape(2 * _N, _D_IN // 2)
    ids_col = node_graph_ids.astype(f32)[:, None]
    w1t = W1.reshape((_MAX_DEG + 1) * _H, _D_IN).T.astype(jnp.bfloat16)
    b1r = b1.reshape(1, (_MAX_DEG + 1) * _H)
    w2t = W2.reshape((_MAX_DEG + 1) * _H, _H).T.astype(jnp.bfloat16)
    b2r = b2.reshape(1, (_MAX_DEG + 1) * _H)
    wngt = W_ng.T
    bngr = b_ng.reshape(1, _PRED_H)
    woutt = W_out.T
    boutr = b_out.reshape(1, 1)
    g1 = gamma1.reshape(1, _H)
    be1 = beta1.reshape(1, _H)
    g2 = gamma2.reshape(1, _H)
    be2 = beta2.reshape(1, _H)
    gp = gamma_p.reshape(1, _PRED_H)
    bep = beta_p.reshape(1, _PRED_H)

    agg1p, degp = _make_sc_agg1()(table1, src3a, dst3a)

    wide = (_MAX_DEG + 1) * _H
    h1p, st1 = pl.pallas_call(
        _k1_body,
        grid=(_NB,),
        in_specs=[
            _node_spec(_D_IN), _node_spec(_H), _node_spec(_H),
            pl.BlockSpec((_NS, _BLK), lambda i: (0, i)),
            _const_spec((_D_IN, wide)), _const_spec((1, wide)),
        ],
        out_specs=[_node_spec(_H), _const_spec((2, _H))],
        out_shape=[jax.ShapeDtypeStruct((_N, _H), f32),
                   jax.ShapeDtypeStruct((2, _H), f32)],
    )(feats, agg1p[0], agg1p[1], degp, w1t, b1r)

    agg2p = _make_sc_agg2()(h1p, src3, dst3)[0]

    h2p, st2 = pl.pallas_call(
        _k2_body,
        grid=(_NB,),
        in_specs=[
            _node_spec(_H), _node_spec(_H), _node_spec(_H),
            pl.BlockSpec((_NS, _BLK), lambda i: (0, i)),
            _const_spec((2, _H)), _const_spec((1, _H)), _const_spec((1, _H)),
            _const_spec((_H, wide)), _const_spec((1, wide)),
        ],
        out_specs=[_node_spec(_H), _const_spec((2, _H))],
        out_shape=[jax.ShapeDtypeStruct((_N, _H), f32),
                   jax.ShapeDtypeStruct((2, _H), f32)],
    )(h1p, agg2p[0], agg2p[1], degp, st1, g1, be1, w2t, b2r)

    psum, pcnt, pmax, pmin, st3 = pl.pallas_call(
        _k3_body,
        grid=(_NB,),
        in_specs=[
            _node_spec(_H), _node_spec(1),
            _const_spec((2, _H)), _const_spec((1, _H)), _const_spec((1, _H)),
            _const_spec((_H, _PRED_H)), _const_spec((1, _PRED_H)),
        ],
        out_specs=[_const_spec((_B, _PRED_H)), _const_spec((_B, 1)),
                   _const_spec((_B, _PRED_H)), _const_spec((_B, _PRED_H)),
                   _const_spec((2, _PRED_H))],
        out_shape=[jax.ShapeDtypeStruct((_B, _PRED_H), f32),
                   jax.ShapeDtypeStruct((_B, 1), f32),
                   jax.ShapeDtypeStruct((_B, _PRED_H), f32),
                   jax.ShapeDtypeStruct((_B, _PRED_H), f32),
                   jax.ShapeDtypeStruct((2, _PRED_H), f32)],
    )(h2p, ids_col, st2, g2, be2, wngt, bngr)

    out = pl.pallas_call(
        _k4_body,
        out_shape=jax.ShapeDtypeStruct((_B, 1), f32),
    )(psum, pcnt, pmax, pmin, st3, gp, bep, woutt, boutr)
    return out


# speedup vs baseline: 1.1110x; 1.0096x over previous
import functools

import jax
import jax.numpy as jnp
from jax import lax
from jax.experimental import pallas as pl
from jax.experimental.pallas import tpu as pltpu
from jax.experimental.pallas import tpu_sc as plsc

_N = 10000
_E = 320000
_D_IN = 128
_H = 64
_MAX_DEG = 10
_PRED_H = 128
_B = 64

_NC = 2
_NS = 16
_NW = _NC * _NS
_EW = _E // _NW
_CH = 80
_NCH = _EW // _CH
_RT1 = _N // _NS

_BLK = 512
_NB = (_N + _BLK - 1) // _BLK


def _sc_pipeline(table, srcv, dstv, rowsv, accsh, gsems, ssems, nch,
                 hook=None):
    def start_gather(i, b):
        pltpu.async_copy(table.at[srcv.at[i]], rowsv.at[b], gsems[b])

    def wait_gather(i, b):
        pltpu.make_async_copy(table.at[srcv.at[i]], rowsv.at[b],
                              gsems[b]).wait()

    def start_scatter(i, b):
        pltpu.async_copy(rowsv.at[b], accsh.at[dstv.at[i]], ssems[b],
                         add=True)

    def wait_scatter(i, b):
        pltpu.make_async_copy(rowsv.at[b], accsh.at[dstv.at[i]],
                              ssems[b]).wait()

    start_gather(0, 0)
    start_gather(1, 1)
    wait_gather(0, 0)
    start_scatter(0, 0)
    start_gather(2, 2)
    if hook is not None:
        hook(0)
    wait_gather(1, 1)
    start_scatter(1, 1)
    start_gather(3, 3)
    if hook is not None:
        hook(1)

    def group(g, _):
        for b in range(4):
            i = g * 4 + b + 2
            bb = (b + 2) % 4

            @pl.when(i < nch)
            def _():
                wait_gather(i, bb)
                start_scatter(i, bb)
                if hook is not None:
                    hook(i)

            @pl.when(i - 2 < nch)
            def _():
                wait_scatter(i - 2, b)

                @pl.when(i + 2 < nch)
                def _():
                    start_gather(i + 2, b)
        return 0

    lax.fori_loop(0, (nch + 3) // 4, group, 0)


def _sc_zero_acc(rowsv0, accsh, s):
    z16 = jnp.zeros((16,), jnp.float32)
    d = rowsv0.shape[1]

    def zr(r, _):
        for k in range(d // 16):
            rowsv0[r, pl.ds(k * 16, 16)] = z16
        return 0

    lax.fori_loop(0, _CH, zr, 0)
    for q in range(_RT1 // _CH):
        pltpu.sync_copy(rowsv0, accsh.at[pl.ds(s * _RT1 + q * _CH, _CH)])
    rem = _RT1 % _CH
    if rem:
        pltpu.sync_copy(
            rowsv0.at[pl.ds(0, rem)],
            accsh.at[pl.ds(s * _RT1 + _RT1 - rem, rem)])


def _sc_zero_deg(degv):
    z16 = jnp.zeros((16,), jnp.float32)

    def zd(j, _):
        degv[pl.ds(j * 16, 16)] = z16
        return 0

    lax.fori_loop(0, _N // 16, zd, 0)


def _sc_count_deg(dstv, degv, nch):
    ones16 = jnp.full((16,), 1.0, jnp.float32)

    def deg_row(j, _):
        for k in range(_CH // 16):
            d16 = dstv[j, pl.ds(k * 16, 16)]
            plsc.addupdate_scatter(degv, [d16], ones16)
        return 0

    lax.fori_loop(0, nch, deg_row, 0)


_SC_PARAMS = pltpu.CompilerParams(needs_layout_passes=False,
                                  use_tc_tiling_on_sc=False)
_NCH1 = _E // _NS // _CH


def _make_sc_agg1():
    mesh = plsc.VectorSubcoreMesh(core_axis_name="c", subcore_axis_name="s")
    D = _D_IN // 2
    out_type = [jax.ShapeDtypeStruct((_NC, _N, D), jnp.float32),
                jax.ShapeDtypeStruct((_NS, _N), jnp.float32)]
    scratch = [
        pltpu.VMEM((_NCH1, _CH), jnp.int32),
        pltpu.VMEM((_NCH1, _CH), jnp.int32),
        pltpu.VMEM((_NCH1, _CH), jnp.int32),
        pltpu.VMEM((4, _CH, D), jnp.float32),
        pltpu.VMEM((_N,), jnp.float32),
        pltpu.VMEM_SHARED((_N, D), jnp.float32),
        [pltpu.SemaphoreType.DMA] * 4,
        [pltpu.SemaphoreType.DMA] * 4,
    ]

    def body(table, src3, dst3, agg_out, deg_out,
             srcv, srci, dstv, rowsv, degv, accsh, gsems, ssems):
        c = lax.axis_index("c")
        s = lax.axis_index("s")

        pltpu.sync_copy(src3.at[s], srcv)
        pltpu.sync_copy(dst3.at[s], dstv)

        def dec_row(r, _):
            for k in range(_CH // 16):
                v = srcv[r, pl.ds(k * 16, 16)]
                srci[r, pl.ds(k * 16, 16)] = v * 2 + c
            return 0

        lax.fori_loop(0, _NCH1, dec_row, 0)
        _sc_zero_acc(rowsv.at[0], accsh, s)

        @pl.when(c == 0)
        def _():
            _sc_zero_deg(degv)

        ones16 = jnp.full((16,), 1.0, jnp.float32)

        def deg_hook(i):
            @pl.when(c == 0)
            def _():
                for k in range(_CH // 16):
                    d16 = dstv[i, pl.ds(k * 16, 16)]
                    plsc.addupdate_scatter(degv, [d16], ones16)

        plsc.subcore_barrier()
        _sc_pipeline(table, srci, dstv, rowsv, accsh, gsems, ssems, _NCH1,
                     hook=deg_hook)
        plsc.subcore_barrier()

        pltpu.sync_copy(accsh.at[pl.ds(s * _RT1, _RT1)],
                        agg_out.at[c, pl.ds(s * _RT1, _RT1)])

        @pl.when(c == 0)
        def _():
            pltpu.sync_copy(degv, deg_out.at[s])

    return pl.kernel(body, out_type=out_type, mesh=mesh,
                     scratch_types=scratch, compiler_params=_SC_PARAMS)


def _make_sc_agg2():
    mesh = plsc.VectorSubcoreMesh(core_axis_name="c", subcore_axis_name="s")
    out_type = [jax.ShapeDtypeStruct((_NC, _N, _H), jnp.float32)]
    scratch = [
        pltpu.VMEM((_NCH, _CH), jnp.int32),
        pltpu.VMEM((_NCH, _CH), jnp.int32),
        pltpu.VMEM((4, _CH, _H), jnp.float32),
        pltpu.VMEM_SHARED((_N, _H), jnp.float32),
        [pltpu.SemaphoreType.DMA] * 4,
        [pltpu.SemaphoreType.DMA] * 4,
    ]

    def body(table, src3, dst3, agg_out,
             srcv, dstv, rowsv, accsh, gsems, ssems):
        c = lax.axis_index("c")
        s = lax.axis_index("s")
        wid = s * _NC + c

        pltpu.sync_copy(src3.at[wid], srcv)
        pltpu.sync_copy(dst3.at[wid], dstv)
        _sc_zero_acc(rowsv.at[0], accsh, s)

        plsc.subcore_barrier()
        _sc_pipeline(table, srcv, dstv, rowsv, accsh, gsems, ssems, _NCH)
        plsc.subcore_barrier()

        pltpu.sync_copy(accsh.at[pl.ds(s * _RT1, _RT1)],
                        agg_out.at[c, pl.ds(s * _RT1, _RT1)])

    return pl.kernel(body, out_type=out_type, mesh=mesh,
                     scratch_types=scratch, compiler_params=_SC_PARAMS)


def _valid_mask(i):
    return (i * _BLK + lax.broadcasted_iota(jnp.int32, (_BLK, 1), 0)) < _N


def _deg_col(degp):
    ones = jnp.ones((_NS, 1), jnp.float32)
    return lax.dot_general(degp, ones, (((0,), (0,)), ((), ())),
                           preferred_element_type=jnp.float32)


def _bank_select(hwide, deg_col, width):
    degc = jnp.minimum(deg_col, float(_MAX_DEG))
    acc = jnp.zeros((_BLK, width), jnp.float32)
    for d in range(_MAX_DEG + 1):
        sel = degc == float(d)
        acc = acc + jnp.where(sel, hwide[:, d * width:(d + 1) * width], 0.0)
    return acc


def _affine(stats, gamma, beta):
    mean = stats[0:1, :] / float(_N)
    var = stats[1:2, :] / float(_N) - mean * mean
    a = gamma * lax.rsqrt(var + 1e-5)
    c = beta - mean * a
    return a, c


def _k1_body(feats, a0, a1, degp, w, b, h_out, st_out):
    i = pl.program_id(0)
    x = feats[...] + jnp.concatenate([a0[...], a1[...]], axis=1)
    deg = _deg_col(degp[...])
    hwide = jnp.dot(x.astype(jnp.bfloat16), w[...],
                    preferred_element_type=jnp.float32) + b[...]
    y = jnp.maximum(_bank_select(hwide, deg, _H), 0.0)
    valid = _valid_mask(i)
    ym = jnp.where(valid, y, 0.0)
    h_out[...] = y

    @pl.when(i == 0)
    def _():
        st_out[...] = jnp.zeros_like(st_out)

    st_out[...] += jnp.concatenate(
        [jnp.sum(ym, axis=0, keepdims=True),
         jnp.sum(jnp.where(valid, y * y, 0.0), axis=0, keepdims=True)], axis=0)


def _k2_body(h1, a0, a1, degp, st1, gamma1, beta1, w, b, h_out, st_out):
    i = pl.program_id(0)
    a_, c_ = _affine(st1[...], gamma1[...], beta1[...])
    deg = _deg_col(degp[...])
    x = a_ * (h1[...] + a0[...] + a1[...]) + c_ * (1.0 + deg)
    hwide = jnp.dot(x.astype(jnp.bfloat16), w[...],
                    preferred_element_type=jnp.float32) + b[...]
    y = jnp.maximum(_bank_select(hwide, deg, _H), 0.0)
    valid = _valid_mask(i)
    ym = jnp.where(valid, y, 0.0)
    h_out[...] = y

    @pl.when(i == 0)
    def _():
        st_out[...] = jnp.zeros_like(st_out)

    st_out[...] += jnp.concatenate(
        [jnp.sum(ym, axis=0, keepdims=True),
         jnp.sum(jnp.where(valid, y * y, 0.0), axis=0, keepdims=True)], axis=0)


def _k3_body(h2, ids, st2, gamma2, beta2, w, b,
             psum, pcnt, pmax, pmin, st_out):
    i = pl.program_id(0)
    a_, c_ = _affine(st2[...], gamma2[...], beta2[...])
    z = a_ * h2[...] + c_
    h3 = jnp.dot(z, w[...], preferred_element_type=jnp.float32) + b[...]
    valid = _valid_mask(i)
    h3 = jnp.where(valid, h3, 0.0)

    @pl.when(i == 0)
    def _():
        psum[...] = jnp.zeros_like(psum)
        pcnt[...] = jnp.zeros_like(pcnt)
        pmax[...] = jnp.full_like(pmax, -jnp.inf)
        pmin[...] = jnp.full_like(pmin, jnp.inf)
        st_out[...] = jnp.zeros_like(st_out)

    st_out[...] += jnp.concatenate(
        [jnp.sum(h3, axis=0, keepdims=True),
         jnp.sum(jnp.where(valid, h3 * h3, 0.0), axis=0, keepdims=True)],
        axis=0)

    idc = ids[...]
    onehot = jnp.where(
        (idc == lax.broadcasted_iota(jnp.int32, (1, _B), 1).astype(jnp.float32))
        & valid,
        1.0, 0.0)
    psum[...] += lax.dot_general(onehot, h3, (((0,), (0,)), ((), ())),
                                 preferred_element_type=jnp.float32)
    pcnt[...] += lax.dot_general(onehot, jnp.where(valid, 1.0, 0.0),
                                 (((0,), (0,)), ((), ())),
                                 preferred_element_type=jnp.float32)

    lo = jnp.min(jnp.where(valid, idc, float(_B))).astype(jnp.int32)
    hi = jnp.max(jnp.where(valid, idc, -1.0)).astype(jnp.int32)

    def mbody(g, _):
        m = (idc == g.astype(jnp.float32)) & valid
        row = lax.broadcasted_iota(jnp.int32, (_B, 1), 0) == g
        cmx = jnp.max(jnp.where(m, h3, -jnp.inf), axis=0, keepdims=True)
        cmn = jnp.min(jnp.where(m, h3, jnp.inf), axis=0, keepdims=True)
        pmax[...] = jnp.where(row, jnp.maximum(pmax[...], cmx), pmax[...])
        pmin[...] = jnp.where(row, jnp.minimum(pmin[...], cmn), pmin[...])
        return 0

    lax.fori_loop(lo, hi + 1, mbody, 0)


def _k4_body(psum, pcnt, pmax, pmin, st3, gamma_p, beta_p, w, b, out):
    a_, c_ = _affine(st3[...], gamma_p[...], beta_p[...])
    sum_n = a_ * psum[...] + c_ * pcnt[...]
    max_n = jnp.where(a_ >= 0.0, a_ * pmax[...] + c_, a_ * pmin[...] + c_)
    g = jnp.tanh(jnp.concatenate([sum_n, max_n], axis=1))
    out[...] = jnp.dot(g, w[...], preferred_element_type=jnp.float32) + b[...]


def _node_spec(width):
    return pl.BlockSpec((_BLK, width), lambda i: (i, 0))


def _const_spec(shape):
    return pl.BlockSpec(shape, lambda i: tuple(0 for _ in shape))


def kernel(feats, edge_index, node_graph_ids, W1, b1, gamma1, beta1,
           W2, b2, gamma2, beta2, W_ng, b_ng, gamma_p, beta_p, W_out, b_out):
    f32 = jnp.float32
    src = edge_index[0]
    dst = edge_index[1]
    src3 = src.reshape(_NW, _NCH, _CH)
    dst3 = dst.reshape(_NW, _NCH, _CH)
    src3a = src.reshape(_NS, _NCH1, _CH)
    dst3a = dst.reshape(_NS, _NCH1, _CH)
    table1 = feats.reshape(2 * _N, _D_IN // 2)
    ids_col = node_graph_ids.astype(f32)[:, None]
    w1t = W1.reshape((_MAX_DEG + 1) * _H, _D_IN).T.astype(jnp.bfloat16)
    b1r = b1.reshape(1, (_MAX_DEG + 1) * _H)
    w2t = W2.reshape((_MAX_DEG + 1) * _H, _H).T.astype(jnp.bfloat16)
    b2r = b2.reshape(1, (_MAX_DEG + 1) * _H)
    wngt = W_ng.T
    bngr = b_ng.reshape(1, _PRED_H)
    woutt = W_out.T
    boutr = b_out.reshape(1, 1)
    g1 = gamma1.reshape(1, _H)
    be1 = beta1.reshape(1, _H)
    g2 = gamma2.reshape(1, _H)
    be2 = beta2.reshape(1, _H)
    gp = gamma_p.reshape(1, _PRED_H)
    bep = beta_p.reshape(1, _PRED_H)

    agg1p, degp = _make_sc_agg1()(table1, src3a, dst3a)

    wide = (_MAX_DEG + 1) * _H
    h1p, st1 = pl.pallas_call(
        _k1_body,
        grid=(_NB,),
        in_specs=[
            _node_spec(_D_IN), _node_spec(_H), _node_spec(_H),
            pl.BlockSpec((_NS, _BLK), lambda i: (0, i)),
            _const_spec((_D_IN, wide)), _const_spec((1, wide)),
        ],
        out_specs=[_node_spec(_H), _const_spec((2, _H))],
        out_shape=[jax.ShapeDtypeStruct((_N, _H), f32),
                   jax.ShapeDtypeStruct((2, _H), f32)],
    )(feats, agg1p[0], agg1p[1], degp, w1t, b1r)

    agg2p = _make_sc_agg2()(h1p, src3, dst3)[0]

    h2p, st2 = pl.pallas_call(
        _k2_body,
        grid=(_NB,),
        in_specs=[
            _node_spec(_H), _node_spec(_H), _node_spec(_H),
            pl.BlockSpec((_NS, _BLK), lambda i: (0, i)),
            _const_spec((2, _H)), _const_spec((1, _H)), _const_spec((1, _H)),
            _const_spec((_H, wide)), _const_spec((1, wide)),
        ],
        out_specs=[_node_spec(_H), _const_spec((2, _H))],
        out_shape=[jax.ShapeDtypeStruct((_N, _H), f32),
                   jax.ShapeDtypeStruct((2, _H), f32)],
    )(h1p, agg2p[0], agg2p[1], degp, st1, g1, be1, w2t, b2r)

    psum, pcnt, pmax, pmin, st3 = pl.pallas_call(
        _k3_body,
        grid=(_NB,),
        in_specs=[
            _node_spec(_H), _node_spec(1),
            _const_spec((2, _H)), _const_spec((1, _H)), _const_spec((1, _H)),
            _const_spec((_H, _PRED_H)), _const_spec((1, _PRED_H)),
        ],
        out_specs=[_const_spec((_B, _PRED_H)), _const_spec((_B, 1)),
                   _const_spec((_B, _PRED_H)), _const_spec((_B, _PRED_H)),
                   _const_spec((2, _PRED_H))],
        out_shape=[jax.ShapeDtypeStruct((_B, _PRED_H), f32),
                   jax.ShapeDtypeStruct((_B, 1), f32),
                   jax.ShapeDtypeStruct((_B, _PRED_H), f32),
                   jax.ShapeDtypeStruct((_B, _PRED_H), f32),
                   jax.ShapeDtypeStruct((2, _PRED_H), f32)],
    )(h2p, ids_col, st2, g2, be2, wngt, bngr)

    out = pl.pallas_call(
        _k4_body,
        out_shape=jax.ShapeDtypeStruct((_B, 1), f32),
    )(psum, pcnt, pmax, pmin, st3, gp, bep, woutt, boutr)
    return out
